# baseline (speedup 1.0000x reference)
import functools

import jax
import jax.numpy as jnp
from jax import lax
from jax.experimental import pallas as pl
from jax.experimental.pallas import tpu as pltpu
from jax.experimental.pallas import tpu_sc as plsc

_MAX_ID = 100000
_ROWS = _MAX_ID + 1
_LANE_BLK = 8192
_V_PAD = 13 * _LANE_BLK


def _mlp_body(tab_ref, w1_ref, b1_ref, w2_ref, b2_ref, out_ref):
    h = lax.dot_general(w1_ref[...], tab_ref[...], (((1,), (1,)), ((), ())),
                        preferred_element_type=jnp.float32)
    h = jnp.maximum(h + b1_ref[...][:, 0:1], 0.0)
    z = lax.dot_general(w2_ref[...], h, (((1,), (0,)), ((), ())),
                        preferred_element_type=jnp.float32)
    out_ref[...] = jax.nn.sigmoid(z + b2_ref[...][:, 0:1])[0]


def _row_vals(table, W1, b1, W2, b2):
    out = pl.pallas_call(
        _mlp_body,
        grid=(_V_PAD // _LANE_BLK,),
        in_specs=[
            pl.BlockSpec((_LANE_BLK, 16), lambda i: (i, 0)),
            pl.BlockSpec((8, 16), lambda i: (0, 0)),
            pl.BlockSpec((8, 128), lambda i: (0, 0)),
            pl.BlockSpec((1, 8), lambda i: (0, 0)),
            pl.BlockSpec((1, 128), lambda i: (0, 0)),
        ],
        out_specs=pl.BlockSpec((_LANE_BLK,), lambda i: (i,)),
        out_shape=jax.ShapeDtypeStruct((_V_PAD,), jnp.float32),
    )(table, W1, jnp.broadcast_to(b1[:, None], (8, 128)),
      W2, jnp.broadcast_to(b2[:, None], (1, 128)))
    return out


@functools.lru_cache(maxsize=None)
def _gather_kernel(nrows, ncols):
    info = plsc.get_sparse_core_info()
    nc, ns = info.num_cores, info.num_subcores
    nw = nc * ns
    rows_per_tile = nrows // nw
    rc = 16
    n_pairs = rows_per_tile // (2 * rc)
    starts = list(range(0, ncols - 15, 16))
    if starts[-1] + 16 < ncols:
        starts.append(ncols - 16)
    mesh = plsc.VectorSubcoreMesh(core_axis_name="c", subcore_axis_name="s")

    @functools.partial(
        pl.kernel, mesh=mesh,
        out_type=jax.ShapeDtypeStruct((nrows, ncols), jnp.float32),
        compiler_params=pltpu.CompilerParams(needs_layout_passes=False),
        scratch_types=[
            pltpu.VMEM((_V_PAD,), jnp.float32),
            pltpu.VMEM((rc, ncols), jnp.int32),
            pltpu.VMEM((rc, ncols), jnp.int32),
            pltpu.VMEM((rc, ncols), jnp.float32),
            pltpu.VMEM((rc, ncols), jnp.float32),
            pltpu.SemaphoreType.DMA,
            pltpu.SemaphoreType.DMA,
            pltpu.SemaphoreType.DMA,
            pltpu.SemaphoreType.DMA,
            pltpu.SemaphoreType.DMA,
        ],
    )
    def gather_k(vals_hbm, ids_hbm, out_hbm, vals_v, idx0, idx1, out0, out1,
                 sv, si0, si1, so0, so1):
        wid = lax.axis_index("s") * nc + lax.axis_index("c")
        base = wid * rows_per_tile
        vcp = pltpu.async_copy(vals_hbm, vals_v, sv)
        pltpu.async_copy(ids_hbm.at[pl.ds(base, rc), :], idx0, si0)
        vcp.wait()

        def do_chunk(row, idx_v, out_v, so):
            @plsc.parallel_loop(0, rc, step=1, unroll=2)
            def _row(i):
                for c16 in starts:
                    ids16 = idx_v[i, pl.ds(c16, 16)]
                    ids16 = jnp.minimum(jnp.maximum(ids16, 0), _MAX_ID)
                    out_v[i, pl.ds(c16, 16)] = plsc.load_gather(vals_v, [ids16])

            pltpu.async_copy(out_v, out_hbm.at[pl.ds(row, rc), :], so)

        def pair_body(p, carry):
            r0 = base + 2 * p * rc
            r1 = r0 + rc
            pltpu.async_copy(ids_hbm.at[pl.ds(r1, rc), :], idx1, si1)
            pltpu.make_async_copy(ids_hbm.at[pl.ds(r0, rc), :], idx0, si0).wait()

            @pl.when(p > 0)
            def _():
                pltpu.make_async_copy(
                    out0, out_hbm.at[pl.ds(r0 - 2 * rc, rc), :], so0).wait()

            do_chunk(r0, idx0, out0, so0)

            @pl.when(p < n_pairs - 1)
            def _():
                pltpu.async_copy(ids_hbm.at[pl.ds(r1 + rc, rc), :], idx0, si0)

            pltpu.make_async_copy(ids_hbm.at[pl.ds(r1, rc), :], idx1, si1).wait()

            @pl.when(p > 0)
            def _():
                pltpu.make_async_copy(
                    out1, out_hbm.at[pl.ds(r1 - 2 * rc, rc), :], so1).wait()

            do_chunk(r1, idx1, out1, so1)
            return carry

        lax.fori_loop(0, n_pairs, pair_body, 0)
        last0 = base + 2 * (n_pairs - 1) * rc
        pltpu.make_async_copy(out0, out_hbm.at[pl.ds(last0, rc), :], so0).wait()
        pltpu.make_async_copy(out1, out_hbm.at[pl.ds(last0 + rc, rc), :], so1).wait()

    return gather_k


def kernel(league_ids, table, W1, b1, W2, b2):
    vals = _row_vals(table, W1, b1, W2, b2)
    nrows, ncols = league_ids.shape
    return _gather_kernel(nrows, ncols)(vals, league_ids)


# speedup vs baseline: 1.3877x; 1.3877x over previous
import functools

import jax
import jax.numpy as jnp
from jax import lax
from jax.experimental import pallas as pl
from jax.experimental.pallas import tpu as pltpu
from jax.experimental.pallas import tpu_sc as plsc

_MAX_ID = 100000
_ROWS = _MAX_ID + 1
_LANE_BLK = 8192
_V_PAD = 13 * _LANE_BLK


def _mlp_body(tT_ref, w1_ref, b1_ref, w2_ref, b2_ref, out_ref):
    h = lax.dot_general(w1_ref[...], tT_ref[...], (((1,), (0,)), ((), ())),
                        preferred_element_type=jnp.float32)
    h = jnp.maximum(h + b1_ref[...][:, 0:1], 0.0)
    z = lax.dot_general(w2_ref[...], h, (((1,), (0,)), ((), ())),
                        preferred_element_type=jnp.float32)
    out_ref[...] = jax.nn.sigmoid(z + b2_ref[...][:, 0:1])[0]


def _row_vals(table, W1, b1, W2, b2):
    out = pl.pallas_call(
        _mlp_body,
        grid=(_V_PAD // _LANE_BLK,),
        in_specs=[
            pl.BlockSpec((16, _LANE_BLK), lambda i: (0, i)),
            pl.BlockSpec((8, 16), lambda i: (0, 0)),
            pl.BlockSpec((8, 128), lambda i: (0, 0)),
            pl.BlockSpec((1, 8), lambda i: (0, 0)),
            pl.BlockSpec((1, 128), lambda i: (0, 0)),
        ],
        out_specs=pl.BlockSpec((_LANE_BLK,), lambda i: (i,)),
        out_shape=jax.ShapeDtypeStruct((_V_PAD,), jnp.float32),
    )(table.T, W1, jnp.broadcast_to(b1[:, None], (8, 128)),
      W2, jnp.broadcast_to(b2[:, None], (1, 128)))
    return out


@functools.lru_cache(maxsize=None)
def _gather_kernel(nrows, ncols):
    info = plsc.get_sparse_core_info()
    nc, ns = info.num_cores, info.num_subcores
    nw = nc * ns
    rows_per_tile = nrows // nw
    rc = 16
    n_pairs = rows_per_tile // (2 * rc)
    starts = list(range(0, ncols - 15, 16))
    if starts[-1] + 16 < ncols:
        starts.append(ncols - 16)
    mesh = plsc.VectorSubcoreMesh(core_axis_name="c", subcore_axis_name="s")

    @functools.partial(
        pl.kernel, mesh=mesh,
        out_type=jax.ShapeDtypeStruct((nrows, ncols), jnp.float32),
        compiler_params=pltpu.CompilerParams(needs_layout_passes=False),
        scratch_types=[
            pltpu.VMEM((_V_PAD,), jnp.float32),
            pltpu.VMEM((rc, ncols), jnp.int32),
            pltpu.VMEM((rc, ncols), jnp.int32),
            pltpu.VMEM((rc, ncols), jnp.float32),
            pltpu.VMEM((rc, ncols), jnp.float32),
            pltpu.SemaphoreType.DMA,
            pltpu.SemaphoreType.DMA,
            pltpu.SemaphoreType.DMA,
            pltpu.SemaphoreType.DMA,
            pltpu.SemaphoreType.DMA,
        ],
    )
    def gather_k(vals_hbm, ids_hbm, out_hbm, vals_v, idx0, idx1, out0, out1,
                 sv, si0, si1, so0, so1):
        wid = lax.axis_index("s") * nc + lax.axis_index("c")
        base = wid * rows_per_tile
        vcp = pltpu.async_copy(vals_hbm, vals_v, sv)
        pltpu.async_copy(ids_hbm.at[pl.ds(base, rc), :], idx0, si0)
        vcp.wait()

        def do_chunk(row, idx_v, out_v, so):
            @plsc.parallel_loop(0, rc, step=1, unroll=4)
            def _row(i):
                for c16 in starts:
                    ids16 = idx_v[i, pl.ds(c16, 16)]
                    ids16 = jnp.minimum(jnp.maximum(ids16, 0), _MAX_ID)
                    out_v[i, pl.ds(c16, 16)] = plsc.load_gather(vals_v, [ids16])

            pltpu.async_copy(out_v, out_hbm.at[pl.ds(row, rc), :], so)

        def pair_body(p, carry):
            r0 = base + 2 * p * rc
            r1 = r0 + rc
            pltpu.async_copy(ids_hbm.at[pl.ds(r1, rc), :], idx1, si1)
            pltpu.make_async_copy(ids_hbm.at[pl.ds(r0, rc), :], idx0, si0).wait()

            @pl.when(p > 0)
            def _():
                pltpu.make_async_copy(
                    out0, out_hbm.at[pl.ds(r0 - 2 * rc, rc), :], so0).wait()

            do_chunk(r0, idx0, out0, so0)

            @pl.when(p < n_pairs - 1)
            def _():
                pltpu.async_copy(ids_hbm.at[pl.ds(r1 + rc, rc), :], idx0, si0)

            pltpu.make_async_copy(ids_hbm.at[pl.ds(r1, rc), :], idx1, si1).wait()

            @pl.when(p > 0)
            def _():
                pltpu.make_async_copy(
                    out1, out_hbm.at[pl.ds(r1 - 2 * rc, rc), :], so1).wait()

            do_chunk(r1, idx1, out1, so1)
            return carry

        lax.fori_loop(0, n_pairs, pair_body, 0)
        last0 = base + 2 * (n_pairs - 1) * rc
        pltpu.make_async_copy(out0, out_hbm.at[pl.ds(last0, rc), :], so0).wait()
        pltpu.make_async_copy(out1, out_hbm.at[pl.ds(last0 + rc, rc), :], so1).wait()

    return gather_k


def kernel(league_ids, table, W1, b1, W2, b2):
    vals = _row_vals(table, W1, b1, W2, b2)
    nrows, ncols = league_ids.shape
    return _gather_kernel(nrows, ncols)(vals, league_ids)


# speedup vs baseline: 1.4036x; 1.0115x over previous
import functools

import jax
import jax.numpy as jnp
from jax import lax
from jax.experimental import pallas as pl
from jax.experimental.pallas import tpu as pltpu
from jax.experimental.pallas import tpu_sc as plsc

_MAX_ID = 100000
_ROWS = _MAX_ID + 1
_LANE_BLK = 8192
_V_PAD = 13 * _LANE_BLK


def _mlp_body(tT_ref, w1_ref, b1_ref, w2_ref, b2_ref, out_ref):
    h = lax.dot_general(w1_ref[...], tT_ref[...], (((1,), (0,)), ((), ())),
                        preferred_element_type=jnp.float32)
    h = jnp.maximum(h + b1_ref[...][:, 0:1], 0.0)
    z = lax.dot_general(w2_ref[...], h, (((1,), (0,)), ((), ())),
                        preferred_element_type=jnp.float32)
    out_ref[...] = jax.nn.sigmoid(z + b2_ref[...][:, 0:1])[0]


def _row_vals(table, W1, b1, W2, b2):
    out = pl.pallas_call(
        _mlp_body,
        grid=(_V_PAD // _LANE_BLK,),
        in_specs=[
            pl.BlockSpec((16, _LANE_BLK), lambda i: (0, i)),
            pl.BlockSpec((8, 16), lambda i: (0, 0)),
            pl.BlockSpec((8, 128), lambda i: (0, 0)),
            pl.BlockSpec((1, 8), lambda i: (0, 0)),
            pl.BlockSpec((1, 128), lambda i: (0, 0)),
        ],
        out_specs=pl.BlockSpec((_LANE_BLK,), lambda i: (i,)),
        out_shape=jax.ShapeDtypeStruct((_V_PAD,), jnp.float32),
    )(table.T, W1, jnp.broadcast_to(b1[:, None], (8, 128)),
      W2, jnp.broadcast_to(b2[:, None], (1, 128)))
    return out


@functools.lru_cache(maxsize=None)
def _gather_kernel(nrows, ncols):
    info = plsc.get_sparse_core_info()
    nc, ns = info.num_cores, info.num_subcores
    nw = nc * ns
    rows_per_tile = nrows // nw
    rc = 16
    n_pairs = rows_per_tile // (2 * rc)
    starts = list(range(0, ncols - 15, 16))
    if starts[-1] + 16 < ncols:
        starts.append(ncols - 16)
    mesh = plsc.VectorSubcoreMesh(core_axis_name="c", subcore_axis_name="s")

    @functools.partial(
        pl.kernel, mesh=mesh,
        out_type=jax.ShapeDtypeStruct((nrows, ncols), jnp.float32),
        compiler_params=pltpu.CompilerParams(needs_layout_passes=False),
        scratch_types=[
            pltpu.VMEM((_V_PAD,), jnp.float32),
            pltpu.VMEM((rc, ncols), jnp.int32),
            pltpu.VMEM((rc, ncols), jnp.int32),
            pltpu.VMEM((rc, ncols), jnp.float32),
            pltpu.VMEM((rc, ncols), jnp.float32),
            pltpu.SemaphoreType.DMA,
            pltpu.SemaphoreType.DMA,
            pltpu.SemaphoreType.DMA,
            pltpu.SemaphoreType.DMA,
            pltpu.SemaphoreType.DMA,
        ],
    )
    def gather_k(vals_hbm, ids_hbm, out_hbm, vals_v, idx0, idx1, out0, out1,
                 sv, si0, si1, so0, so1):
        wid = lax.axis_index("s") * nc + lax.axis_index("c")
        base = wid * rows_per_tile
        vcp = pltpu.async_copy(vals_hbm, vals_v, sv)
        pltpu.async_copy(ids_hbm.at[pl.ds(base, rc), :], idx0, si0)
        vcp.wait()

        def do_chunk(row, idx_v, out_v, so):
            @plsc.parallel_loop(0, rc, step=1, unroll=2)
            def _row(i):
                for c16 in starts:
                    ids16 = idx_v[i, pl.ds(c16, 16)]
                    ids16 = jnp.minimum(jnp.maximum(ids16, 0), _MAX_ID)
                    out_v[i, pl.ds(c16, 16)] = plsc.load_gather(vals_v, [ids16])

            pltpu.async_copy(out_v, out_hbm.at[pl.ds(row, rc), :], so)

        def pair_body(p, carry):
            r0 = base + 2 * p * rc
            r1 = r0 + rc
            pltpu.async_copy(ids_hbm.at[pl.ds(r1, rc), :], idx1, si1)
            pltpu.make_async_copy(ids_hbm.at[pl.ds(r0, rc), :], idx0, si0).wait()

            @pl.when(p > 0)
            def _():
                pltpu.make_async_copy(
                    out0, out_hbm.at[pl.ds(r0 - 2 * rc, rc), :], so0).wait()

            do_chunk(r0, idx0, out0, so0)

            @pl.when(p < n_pairs - 1)
            def _():
                pltpu.async_copy(ids_hbm.at[pl.ds(r1 + rc, rc), :], idx0, si0)

            pltpu.make_async_copy(ids_hbm.at[pl.ds(r1, rc), :], idx1, si1).wait()

            @pl.when(p > 0)
            def _():
                pltpu.make_async_copy(
                    out1, out_hbm.at[pl.ds(r1 - 2 * rc, rc), :], so1).wait()

            do_chunk(r1, idx1, out1, so1)
            return carry

        lax.fori_loop(0, n_pairs, pair_body, 0)
        last0 = base + 2 * (n_pairs - 1) * rc
        pltpu.make_async_copy(out0, out_hbm.at[pl.ds(last0, rc), :], so0).wait()
        pltpu.make_async_copy(out1, out_hbm.at[pl.ds(last0 + rc, rc), :], so1).wait()

    return gather_k


def kernel(league_ids, table, W1, b1, W2, b2):
    vals = _row_vals(table, W1, b1, W2, b2)
    nrows, ncols = league_ids.shape
    return _gather_kernel(nrows, ncols)(vals, league_ids)
